# Initial kernel scaffold; baseline (speedup 1.0000x reference)
import jax
import jax.numpy as jnp
from jax.experimental import pallas as pl


def kernel(x, edge_index, W1, b1):
    raise NotImplementedError("write your pallas kernel here")


# speedup vs baseline: 9.2891x; 9.2891x over previous
import functools

import jax
import jax.numpy as jnp
from jax import lax
from jax.experimental import pallas as pl
from jax.experimental.pallas import tpu as pltpu
from jax.experimental.pallas import tpu_sc as plsc

N_NODES = 10000
D_FEAT = 256
HALF = 128
SCALE = 1.8

NC = 2
NS = 16
CHUNK = 128
E_PAD = 163840
ROWS_ALL = E_PAD // CHUNK
ROWS_DEG = ROWS_ALL // (NC * NS)
ROWS_PROP = ROWS_ALL // NS
ACC_SP = 10112
ZROWS = ACC_SP // NS
DEG_W = 16

ROW_BLK = 1000


def _deg_body(dst_hbm, ones_hbm, zeros_hbm, cnt_hbm, idx_v, ones_v, cnt_sh):
    c = lax.axis_index("c")
    s = lax.axis_index("s")
    w = c * NS + s
    pltpu.sync_copy(zeros_hbm, cnt_sh.at[pl.ds(s * ZROWS, ZROWS)])
    pltpu.sync_copy(ones_hbm, ones_v)
    pltpu.sync_copy(dst_hbm.at[pl.ds(w * ROWS_DEG, ROWS_DEG)], idx_v)
    plsc.subcore_barrier()

    def body(j, carry):
        pltpu.sync_copy(ones_v, cnt_sh.at[idx_v.at[j]], add=True)
        return carry

    lax.fori_loop(0, ROWS_DEG, body, 0)
    plsc.subcore_barrier()
    pltpu.sync_copy(
        cnt_sh.at[pl.ds(s * ZROWS, ZROWS)],
        cnt_hbm.at[c, pl.ds(s * ZROWS, ZROWS)],
    )


def _deg_call(*args):
    mesh = plsc.VectorSubcoreMesh(
        core_axis_name="c", subcore_axis_name="s",
        num_cores=NC, num_subcores=NS,
    )
    fn = functools.partial(
        pl.kernel,
        out_type=jax.ShapeDtypeStruct((NC, ACC_SP, DEG_W), jnp.float32),
        mesh=mesh,
        scratch_types=[
            pltpu.VMEM((ROWS_DEG, CHUNK), jnp.int32),
            pltpu.VMEM((CHUNK, DEG_W), jnp.float32),
            pltpu.VMEM_SHARED((ACC_SP, DEG_W), jnp.float32),
        ],
    )(_deg_body)
    return fn(*args)


def _prop_body(g2_hbm, src_hbm, dst_hbm, zeros_hbm, acc_hbm,
               srcv, dstv, rows_v, acc_sh, sem):
    c = lax.axis_index("c")
    s = lax.axis_index("s")
    pltpu.sync_copy(zeros_hbm, acc_sh.at[pl.ds(s * ZROWS, ZROWS)])
    pltpu.sync_copy(src_hbm.at[c, pl.ds(s * ROWS_PROP, ROWS_PROP)], srcv)
    pltpu.sync_copy(dst_hbm.at[pl.ds(s * ROWS_PROP, ROWS_PROP)], dstv)
    plsc.subcore_barrier()

    def body(j, carry):
        pltpu.async_copy(g2_hbm.at[srcv.at[j]], rows_v, sem).wait()
        pltpu.sync_copy(rows_v, acc_sh.at[dstv.at[j]], add=True)
        return carry

    lax.fori_loop(0, ROWS_PROP, body, 0)
    plsc.subcore_barrier()
    pltpu.sync_copy(
        acc_sh.at[pl.ds(s * ZROWS, ZROWS)],
        acc_hbm.at[c, pl.ds(s * ZROWS, ZROWS)],
    )


def _prop_call(*args):
    mesh = plsc.VectorSubcoreMesh(
        core_axis_name="c", subcore_axis_name="s",
        num_cores=NC, num_subcores=NS,
    )
    fn = functools.partial(
        pl.kernel,
        out_type=jax.ShapeDtypeStruct((NC, ACC_SP, HALF), jnp.float32),
        mesh=mesh,
        scratch_types=[
            pltpu.VMEM((ROWS_PROP, CHUNK), jnp.int32),
            pltpu.VMEM((ROWS_PROP, CHUNK), jnp.int32),
            pltpu.VMEM((CHUNK, HALF), jnp.float32),
            pltpu.VMEM_SHARED((ACC_SP, HALF), jnp.float32),
            pltpu.SemaphoreType.DMA,
        ],
    )(_prop_body)
    return fn(*args)


def _dense_body(x_ref, w_ref, b_ref, dinv_ref, g3_ref):
    h = lax.dot_general(
        x_ref[...], w_ref[...], (((1,), (1,)), ((), ())),
        preferred_element_type=jnp.float32,
    )
    h = h + b_ref[...]
    norm = jnp.sqrt(jnp.sum(h * h, axis=1, keepdims=True))
    g = h / jnp.maximum(norm, 1e-12) * SCALE
    g = g * dinv_ref[...]
    g3_ref[...] = g.reshape(ROW_BLK, 2, HALF)


def _dense_call(x, W1, b1r, dinv2):
    grid = N_NODES // ROW_BLK
    return pl.pallas_call(
        _dense_body,
        grid=(grid,),
        in_specs=[
            pl.BlockSpec((ROW_BLK, D_FEAT), lambda i: (i, 0)),
            pl.BlockSpec((D_FEAT, D_FEAT), lambda i: (0, 0)),
            pl.BlockSpec((1, D_FEAT), lambda i: (0, 0)),
            pl.BlockSpec((ROW_BLK, 1), lambda i: (i, 0)),
        ],
        out_specs=pl.BlockSpec((ROW_BLK, 2, HALF), lambda i: (i, 0, 0)),
        out_shape=jax.ShapeDtypeStruct((N_NODES, 2, HALF), jnp.float32),
    )(x, W1, b1r, dinv2)


def _combine_body(acc_ref, g3_ref, dinv_ref, out_ref):
    dinv = dinv_ref[...]
    out_ref[:, :HALF] = (acc_ref[0] + g3_ref[:, 0, :]) * dinv
    out_ref[:, HALF:] = (acc_ref[1] + g3_ref[:, 1, :]) * dinv


def _combine_call(acc, g3, dinv2):
    grid = N_NODES // ROW_BLK
    return pl.pallas_call(
        _combine_body,
        grid=(grid,),
        in_specs=[
            pl.BlockSpec((NC, ROW_BLK, HALF), lambda i: (0, i, 0)),
            pl.BlockSpec((ROW_BLK, 2, HALF), lambda i: (i, 0, 0)),
            pl.BlockSpec((ROW_BLK, 1), lambda i: (i, 0)),
        ],
        out_specs=pl.BlockSpec((ROW_BLK, D_FEAT), lambda i: (i, 0)),
        out_shape=jax.ShapeDtypeStruct((N_NODES, D_FEAT), jnp.float32),
    )(acc, g3, dinv2)


def kernel(x, edge_index, W1, b1):
    n_edges = edge_index.shape[1]
    pad = E_PAD - n_edges
    src = edge_index[0]
    dst = edge_index[1]
    src_p = jnp.concatenate([src, jnp.zeros((pad,), jnp.int32)])
    dst_p = jnp.concatenate([dst, jnp.full((pad,), N_NODES, jnp.int32)])
    dst2d = dst_p.reshape(ROWS_ALL, CHUNK)
    src2 = jnp.stack([src_p * 2, src_p * 2 + 1]).reshape(NC, ROWS_ALL, CHUNK)

    ones_a = jnp.ones((CHUNK, DEG_W), jnp.float32)
    zeros_a = jnp.zeros((ZROWS, DEG_W), jnp.float32)
    zeros_c = jnp.zeros((ZROWS, HALF), jnp.float32)

    cnt = _deg_call(dst2d, ones_a, zeros_a)
    deg = cnt[0, :N_NODES, 0] + cnt[1, :N_NODES, 0] + 1.0
    dinv2 = lax.rsqrt(deg)[:, None]

    g3 = _dense_call(x, W1, b1[None, :], dinv2)
    g2 = g3.reshape(2 * N_NODES, HALF)

    acc = _prop_call(g2, src2, dst2d, zeros_c)
    return _combine_call(acc[:, :N_NODES], g3, dinv2)


# speedup vs baseline: 9.6235x; 1.0360x over previous
import functools

import jax
import jax.numpy as jnp
from jax import lax
from jax.experimental import pallas as pl
from jax.experimental.pallas import tpu as pltpu
from jax.experimental.pallas import tpu_sc as plsc

N_NODES = 10000
D_FEAT = 256
HALF = 128
SCALE = 1.8

NC = 2
NS = 16
CHUNK = 128
E_PAD = 163840
ROWS_ALL = E_PAD // CHUNK
ROWS_DEG = ROWS_ALL // (NC * NS)
ROWS_PROP = ROWS_ALL // NS
ACC_SP = 10112
ZROWS = ACC_SP // NS
DEG_W = 16

ROW_BLK = 1000


def _deg_body(dst_hbm, ones_hbm, zeros_hbm, cnt_hbm, idx_v, ones_v, cnt_sh):
    c = lax.axis_index("c")
    s = lax.axis_index("s")
    w = c * NS + s
    pltpu.sync_copy(zeros_hbm, cnt_sh.at[pl.ds(s * ZROWS, ZROWS)])
    pltpu.sync_copy(ones_hbm, ones_v)
    pltpu.sync_copy(dst_hbm.at[pl.ds(w * ROWS_DEG, ROWS_DEG)], idx_v)
    plsc.subcore_barrier()

    def body(j, carry):
        pltpu.sync_copy(ones_v, cnt_sh.at[idx_v.at[j]], add=True)
        return carry

    lax.fori_loop(0, ROWS_DEG, body, 0)
    plsc.subcore_barrier()
    pltpu.sync_copy(
        cnt_sh.at[pl.ds(s * ZROWS, ZROWS)],
        cnt_hbm.at[c, pl.ds(s * ZROWS, ZROWS)],
    )


def _deg_call(*args):
    mesh = plsc.VectorSubcoreMesh(
        core_axis_name="c", subcore_axis_name="s",
        num_cores=NC, num_subcores=NS,
    )
    fn = functools.partial(
        pl.kernel,
        out_type=jax.ShapeDtypeStruct((NC, ACC_SP, DEG_W), jnp.float32),
        mesh=mesh,
        scratch_types=[
            pltpu.VMEM((ROWS_DEG, CHUNK), jnp.int32),
            pltpu.VMEM((CHUNK, DEG_W), jnp.float32),
            pltpu.VMEM_SHARED((ACC_SP, DEG_W), jnp.float32),
        ],
    )(_deg_body)
    return fn(*args)


def _prop_body(g2_hbm, src_hbm, dst_hbm, zeros_hbm, acc_hbm,
               srcv, dstv, rows_a, rows_b, acc_sh, sem_a, sem_b):
    c = lax.axis_index("c")
    s = lax.axis_index("s")
    pltpu.sync_copy(zeros_hbm, acc_sh.at[pl.ds(s * ZROWS, ZROWS)])
    plsc.subcore_barrier()
    hr = ROWS_PROP // 2

    bufs = (rows_a, rows_b)
    sems = (sem_a, sem_b)

    for h in range(2):
        pltpu.sync_copy(
            src_hbm.at[c, pl.ds(s * ROWS_PROP + h * hr, hr)], srcv)
        pltpu.sync_copy(dst_hbm.at[pl.ds(s * ROWS_PROP + h * hr, hr)], dstv)
        def body(i, carry):
            j = 2 * i
            pltpu.async_copy(g2_hbm.at[srcv.at[j]], rows_a, sem_a)
            pltpu.async_copy(g2_hbm.at[srcv.at[j + 1]], rows_b, sem_a)
            pltpu.make_async_copy(
                g2_hbm.at[srcv.at[j]], rows_a, sem_a).wait()
            pltpu.make_async_copy(
                g2_hbm.at[srcv.at[j + 1]], rows_b, sem_a).wait()
            pltpu.sync_copy(rows_a, acc_sh.at[dstv.at[j]], add=True)
            pltpu.sync_copy(rows_b, acc_sh.at[dstv.at[j + 1]], add=True)
            return carry

        lax.fori_loop(0, hr // 2, body, 0)
    plsc.subcore_barrier()
    pltpu.sync_copy(
        acc_sh.at[pl.ds(s * ZROWS, ZROWS)],
        acc_hbm.at[c, pl.ds(s * ZROWS, ZROWS)],
    )


def _prop_call(*args):
    mesh = plsc.VectorSubcoreMesh(
        core_axis_name="c", subcore_axis_name="s",
        num_cores=NC, num_subcores=NS,
    )
    fn = functools.partial(
        pl.kernel,
        out_type=jax.ShapeDtypeStruct((NC, ACC_SP, HALF), jnp.float32),
        mesh=mesh,
        scratch_types=[
            pltpu.VMEM((ROWS_PROP // 2, CHUNK), jnp.int32),
            pltpu.VMEM((ROWS_PROP // 2, CHUNK), jnp.int32),
            pltpu.VMEM((CHUNK, HALF), jnp.float32),
            pltpu.VMEM((CHUNK, HALF), jnp.float32),
            pltpu.VMEM_SHARED((ACC_SP, HALF), jnp.float32),
            pltpu.SemaphoreType.DMA,
            pltpu.SemaphoreType.DMA,
        ],
    )(_prop_body)
    return fn(*args)


def _dense_body(x_ref, w_ref, b_ref, dinv_ref, g3_ref):
    h = lax.dot_general(
        x_ref[...], w_ref[...], (((1,), (1,)), ((), ())),
        preferred_element_type=jnp.float32,
    )
    h = h + b_ref[...]
    norm = jnp.sqrt(jnp.sum(h * h, axis=1, keepdims=True))
    g = h / jnp.maximum(norm, 1e-12) * SCALE
    g = g * dinv_ref[...]
    g3_ref[...] = g.reshape(ROW_BLK, 2, HALF)


def _dense_call(x, W1, b1r, dinv2):
    grid = N_NODES // ROW_BLK
    return pl.pallas_call(
        _dense_body,
        grid=(grid,),
        in_specs=[
            pl.BlockSpec((ROW_BLK, D_FEAT), lambda i: (i, 0)),
            pl.BlockSpec((D_FEAT, D_FEAT), lambda i: (0, 0)),
            pl.BlockSpec((1, D_FEAT), lambda i: (0, 0)),
            pl.BlockSpec((ROW_BLK, 1), lambda i: (i, 0)),
        ],
        out_specs=pl.BlockSpec((ROW_BLK, 2, HALF), lambda i: (i, 0, 0)),
        out_shape=jax.ShapeDtypeStruct((N_NODES, 2, HALF), jnp.float32),
    )(x, W1, b1r, dinv2)


def _combine_body(acc_ref, g3_ref, dinv_ref, out_ref):
    dinv = dinv_ref[...]
    out_ref[:, :HALF] = (acc_ref[0] + g3_ref[:, 0, :]) * dinv
    out_ref[:, HALF:] = (acc_ref[1] + g3_ref[:, 1, :]) * dinv


def _combine_call(acc, g3, dinv2):
    grid = N_NODES // ROW_BLK
    return pl.pallas_call(
        _combine_body,
        grid=(grid,),
        in_specs=[
            pl.BlockSpec((NC, ROW_BLK, HALF), lambda i: (0, i, 0)),
            pl.BlockSpec((ROW_BLK, 2, HALF), lambda i: (i, 0, 0)),
            pl.BlockSpec((ROW_BLK, 1), lambda i: (i, 0)),
        ],
        out_specs=pl.BlockSpec((ROW_BLK, D_FEAT), lambda i: (i, 0)),
        out_shape=jax.ShapeDtypeStruct((N_NODES, D_FEAT), jnp.float32),
    )(acc, g3, dinv2)


def kernel(x, edge_index, W1, b1):
    n_edges = edge_index.shape[1]
    pad = E_PAD - n_edges
    src = edge_index[0]
    dst = edge_index[1]
    src_p = jnp.concatenate([src, jnp.zeros((pad,), jnp.int32)])
    dst_p = jnp.concatenate([dst, jnp.full((pad,), N_NODES, jnp.int32)])
    dst2d = dst_p.reshape(ROWS_ALL, CHUNK)
    src2 = jnp.stack([src_p * 2, src_p * 2 + 1]).reshape(NC, ROWS_ALL, CHUNK)

    ones_a = jnp.ones((CHUNK, DEG_W), jnp.float32)
    zeros_a = jnp.zeros((ZROWS, DEG_W), jnp.float32)
    zeros_c = jnp.zeros((ZROWS, HALF), jnp.float32)

    cnt = _deg_call(dst2d, ones_a, zeros_a)
    deg = cnt[0, :N_NODES, 0] + cnt[1, :N_NODES, 0] + 1.0
    dinv2 = lax.rsqrt(deg)[:, None]

    g3 = _dense_call(x, W1, b1[None, :], dinv2)
    g2 = g3.reshape(2 * N_NODES, HALF)

    acc = _prop_call(g2, src2, dst2d, zeros_c)
    return _combine_call(acc[:, :N_NODES], g3, dinv2)


# speedup vs baseline: 10.0547x; 1.0448x over previous
import functools

import jax
import jax.numpy as jnp
from jax import lax
from jax.experimental import pallas as pl
from jax.experimental.pallas import tpu as pltpu
from jax.experimental.pallas import tpu_sc as plsc

N_NODES = 10000
D_FEAT = 256
HALF = 128
SCALE = 1.8

NC = 2
NS = 16
CHUNK = 128
E_PAD = 163840
ROWS_ALL = E_PAD // CHUNK
ROWS_DEG = ROWS_ALL // (NC * NS)
ROWS_PROP = ROWS_ALL // NS
ACC_SP = 10112
ZROWS = ACC_SP // NS
DEG_W = 16

ROW_BLK = 1000


def _deg_body(dst_hbm, ones_hbm, zeros_hbm, cnt_hbm, idx_v, ones_v, cnt_sh):
    c = lax.axis_index("c")
    s = lax.axis_index("s")
    w = c * NS + s
    pltpu.sync_copy(zeros_hbm, cnt_sh.at[pl.ds(s * ZROWS, ZROWS)])
    pltpu.sync_copy(ones_hbm, ones_v)
    pltpu.sync_copy(dst_hbm.at[pl.ds(w * ROWS_DEG, ROWS_DEG)], idx_v)
    plsc.subcore_barrier()

    def body(j, carry):
        pltpu.sync_copy(ones_v, cnt_sh.at[idx_v.at[j]], add=True)
        return carry

    lax.fori_loop(0, ROWS_DEG, body, 0)
    plsc.subcore_barrier()
    pltpu.sync_copy(
        cnt_sh.at[pl.ds(s * ZROWS, ZROWS)],
        cnt_hbm.at[c, pl.ds(s * ZROWS, ZROWS)],
    )


def _deg_call(*args):
    mesh = plsc.VectorSubcoreMesh(
        core_axis_name="c", subcore_axis_name="s",
        num_cores=NC, num_subcores=NS,
    )
    fn = functools.partial(
        pl.kernel,
        out_type=jax.ShapeDtypeStruct((NC, ACC_SP, DEG_W), jnp.float32),
        mesh=mesh,
        scratch_types=[
            pltpu.VMEM((ROWS_DEG, CHUNK), jnp.int32),
            pltpu.VMEM((CHUNK, DEG_W), jnp.float32),
            pltpu.VMEM_SHARED((ACC_SP, DEG_W), jnp.float32),
        ],
    )(_deg_body)
    return fn(*args)


def _prop_body(g2_hbm, src_hbm, dst_hbm, zeros_hbm, acc_hbm,
               srcv, dstv, rows_a, rows_b, acc_sh,
               sem_a, sem_b, sem_sa, sem_sb):
    c = lax.axis_index("c")
    s = lax.axis_index("s")
    pltpu.sync_copy(zeros_hbm, acc_sh.at[pl.ds(s * ZROWS, ZROWS)])
    plsc.subcore_barrier()
    hr = ROWS_PROP // 2

    bufs = (rows_a, rows_b)
    sems = (sem_a, sem_b)

    for h in range(2):
        pltpu.sync_copy(
            src_hbm.at[c, pl.ds(s * ROWS_PROP + h * hr, hr)], srcv)
        pltpu.sync_copy(dst_hbm.at[pl.ds(s * ROWS_PROP + h * hr, hr)], dstv)
        pltpu.async_copy(g2_hbm.at[srcv.at[0]], rows_a, sem_a)
        pltpu.async_copy(g2_hbm.at[srcv.at[1]], rows_b, sem_b)

        def body(i, carry):
            j = 2 * i
            pltpu.make_async_copy(
                g2_hbm.at[srcv.at[j]], rows_a, sem_a).wait()
            pltpu.async_copy(rows_a, acc_sh.at[dstv.at[j]], sem_sa, add=True)
            pltpu.make_async_copy(
                g2_hbm.at[srcv.at[j + 1]], rows_b, sem_b).wait()
            pltpu.async_copy(
                rows_b, acc_sh.at[dstv.at[j + 1]], sem_sb, add=True)
            pltpu.make_async_copy(
                rows_a, acc_sh.at[dstv.at[j]], sem_sa).wait()
            pltpu.async_copy(g2_hbm.at[srcv.at[j + 2]], rows_a, sem_a)
            pltpu.make_async_copy(
                rows_b, acc_sh.at[dstv.at[j + 1]], sem_sb).wait()
            pltpu.async_copy(g2_hbm.at[srcv.at[j + 3]], rows_b, sem_b)
            return carry

        lax.fori_loop(0, hr // 2 - 1, body, 0)
        for b, (buf, gsem, ssem) in enumerate(
                ((rows_a, sem_a, sem_sa), (rows_b, sem_b, sem_sb))):
            j = hr - 2 + b
            pltpu.make_async_copy(g2_hbm.at[srcv.at[j]], buf, gsem).wait()
            pltpu.sync_copy(buf, acc_sh.at[dstv.at[j]], add=True)
    plsc.subcore_barrier()
    pltpu.sync_copy(
        acc_sh.at[pl.ds(s * ZROWS, ZROWS)],
        acc_hbm.at[c, pl.ds(s * ZROWS, ZROWS)],
    )


def _prop_call(*args):
    mesh = plsc.VectorSubcoreMesh(
        core_axis_name="c", subcore_axis_name="s",
        num_cores=NC, num_subcores=NS,
    )
    fn = functools.partial(
        pl.kernel,
        out_type=jax.ShapeDtypeStruct((NC, ACC_SP, HALF), jnp.float32),
        mesh=mesh,
        scratch_types=[
            pltpu.VMEM((ROWS_PROP // 2, CHUNK), jnp.int32),
            pltpu.VMEM((ROWS_PROP // 2, CHUNK), jnp.int32),
            pltpu.VMEM((CHUNK, HALF), jnp.float32),
            pltpu.VMEM((CHUNK, HALF), jnp.float32),
            pltpu.VMEM_SHARED((ACC_SP, HALF), jnp.float32),
            pltpu.SemaphoreType.DMA,
            pltpu.SemaphoreType.DMA,
            pltpu.SemaphoreType.DMA,
            pltpu.SemaphoreType.DMA,
        ],
    )(_prop_body)
    return fn(*args)


def _dense_body(x_ref, w_ref, b_ref, dinv_ref, g3_ref):
    h = lax.dot_general(
        x_ref[...], w_ref[...], (((1,), (1,)), ((), ())),
        preferred_element_type=jnp.float32,
    )
    h = h + b_ref[...]
    norm = jnp.sqrt(jnp.sum(h * h, axis=1, keepdims=True))
    g = h / jnp.maximum(norm, 1e-12) * SCALE
    g = g * dinv_ref[...]
    g3_ref[...] = g.reshape(ROW_BLK, 2, HALF)


def _dense_call(x, W1, b1r, dinv2):
    grid = N_NODES // ROW_BLK
    return pl.pallas_call(
        _dense_body,
        grid=(grid,),
        in_specs=[
            pl.BlockSpec((ROW_BLK, D_FEAT), lambda i: (i, 0)),
            pl.BlockSpec((D_FEAT, D_FEAT), lambda i: (0, 0)),
            pl.BlockSpec((1, D_FEAT), lambda i: (0, 0)),
            pl.BlockSpec((ROW_BLK, 1), lambda i: (i, 0)),
        ],
        out_specs=pl.BlockSpec((ROW_BLK, 2, HALF), lambda i: (i, 0, 0)),
        out_shape=jax.ShapeDtypeStruct((N_NODES, 2, HALF), jnp.float32),
    )(x, W1, b1r, dinv2)


def _combine_body(acc_ref, g3_ref, dinv_ref, out_ref):
    dinv = dinv_ref[...]
    out_ref[:, :HALF] = (acc_ref[0] + g3_ref[:, 0, :]) * dinv
    out_ref[:, HALF:] = (acc_ref[1] + g3_ref[:, 1, :]) * dinv


def _combine_call(acc, g3, dinv2):
    grid = N_NODES // ROW_BLK
    return pl.pallas_call(
        _combine_body,
        grid=(grid,),
        in_specs=[
            pl.BlockSpec((NC, ROW_BLK, HALF), lambda i: (0, i, 0)),
            pl.BlockSpec((ROW_BLK, 2, HALF), lambda i: (i, 0, 0)),
            pl.BlockSpec((ROW_BLK, 1), lambda i: (i, 0)),
        ],
        out_specs=pl.BlockSpec((ROW_BLK, D_FEAT), lambda i: (i, 0)),
        out_shape=jax.ShapeDtypeStruct((N_NODES, D_FEAT), jnp.float32),
    )(acc, g3, dinv2)


def kernel(x, edge_index, W1, b1):
    n_edges = edge_index.shape[1]
    pad = E_PAD - n_edges
    src = edge_index[0]
    dst = edge_index[1]
    src_p = jnp.concatenate([src, jnp.zeros((pad,), jnp.int32)])
    dst_p = jnp.concatenate([dst, jnp.full((pad,), N_NODES, jnp.int32)])
    dst2d = dst_p.reshape(ROWS_ALL, CHUNK)
    src2 = jnp.stack([src_p * 2, src_p * 2 + 1]).reshape(NC, ROWS_ALL, CHUNK)

    ones_a = jnp.ones((CHUNK, DEG_W), jnp.float32)
    zeros_a = jnp.zeros((ZROWS, DEG_W), jnp.float32)
    zeros_c = jnp.zeros((ZROWS, HALF), jnp.float32)

    cnt = _deg_call(dst2d, ones_a, zeros_a)
    deg = cnt[0, :N_NODES, 0] + cnt[1, :N_NODES, 0] + 1.0
    dinv2 = lax.rsqrt(deg)[:, None]

    g3 = _dense_call(x, W1, b1[None, :], dinv2)
    g2 = g3.reshape(2 * N_NODES, HALF)

    acc = _prop_call(g2, src2, dst2d, zeros_c)
    return _combine_call(acc[:, :N_NODES], g3, dinv2)


# speedup vs baseline: 10.3598x; 1.0303x over previous
import functools

import jax
import jax.numpy as jnp
from jax import lax
from jax.experimental import pallas as pl
from jax.experimental.pallas import tpu as pltpu
from jax.experimental.pallas import tpu_sc as plsc

N_NODES = 10000
D_FEAT = 256
HALF = 128
SCALE = 1.8

NC = 2
NS = 16
CHUNK = 128
E_PAD = 163840
ROWS_ALL = E_PAD // CHUNK
ROWS_DEG = ROWS_ALL // (NC * NS)
ROWS_PROP = ROWS_ALL // NS
ACC_SP = 10112
ZROWS = ACC_SP // NS
DEG_W = 16

PCH = 64
PROWS_ALL = E_PAD // PCH
PROWS_SUB = PROWS_ALL // NS
NBUF = 4

ROW_BLK = 1000


def _deg_body(dst_hbm, ones_hbm, zeros_hbm, cnt_hbm, idx_v, ones_v, cnt_sh):
    c = lax.axis_index("c")
    s = lax.axis_index("s")
    w = c * NS + s
    pltpu.sync_copy(zeros_hbm, cnt_sh.at[pl.ds(s * ZROWS, ZROWS)])
    pltpu.sync_copy(ones_hbm, ones_v)
    pltpu.sync_copy(dst_hbm.at[pl.ds(w * ROWS_DEG, ROWS_DEG)], idx_v)
    plsc.subcore_barrier()

    def body(j, carry):
        pltpu.sync_copy(ones_v, cnt_sh.at[idx_v.at[j]], add=True)
        return carry

    lax.fori_loop(0, ROWS_DEG, body, 0)
    plsc.subcore_barrier()
    pltpu.sync_copy(
        cnt_sh.at[pl.ds(s * ZROWS, ZROWS)],
        cnt_hbm.at[c, pl.ds(s * ZROWS, ZROWS)],
    )


def _deg_call(*args):
    mesh = plsc.VectorSubcoreMesh(
        core_axis_name="c", subcore_axis_name="s",
        num_cores=NC, num_subcores=NS,
    )
    fn = functools.partial(
        pl.kernel,
        out_type=jax.ShapeDtypeStruct((NC, ACC_SP, DEG_W), jnp.float32),
        mesh=mesh,
        scratch_types=[
            pltpu.VMEM((ROWS_DEG, CHUNK), jnp.int32),
            pltpu.VMEM((CHUNK, DEG_W), jnp.float32),
            pltpu.VMEM_SHARED((ACC_SP, DEG_W), jnp.float32),
        ],
    )(_deg_body)
    return fn(*args)


def _prop_body(g2_hbm, src_hbm, dst_hbm, zeros_hbm, acc_hbm,
               srcv, dstv, b0, b1, b2, b3, acc_sh,
               g0, g1, g2s, g3s, s0, s1, s2, s3):
    c = lax.axis_index("c")
    s = lax.axis_index("s")
    pltpu.sync_copy(zeros_hbm, acc_sh.at[pl.ds(s * ZROWS, ZROWS)])
    plsc.subcore_barrier()
    hr = PROWS_SUB // 4

    bufs = (b0, b1, b2, b3)
    gsems = (g0, g1, g2s, g3s)
    ssems = (s0, s1, s2, s3)

    for h in range(4):
        pltpu.sync_copy(
            src_hbm.at[c, pl.ds(s * PROWS_SUB + h * hr, hr)], srcv)
        pltpu.sync_copy(dst_hbm.at[pl.ds(s * PROWS_SUB + h * hr, hr)], dstv)
        for b in range(NBUF):
            pltpu.async_copy(g2_hbm.at[srcv.at[b]], bufs[b], gsems[b])

        def body(i, carry):
            j0 = NBUF * i
            for b in range(NBUF):
                pltpu.make_async_copy(
                    g2_hbm.at[srcv.at[j0 + b]], bufs[b], gsems[b]).wait()
                pltpu.async_copy(
                    bufs[b], acc_sh.at[dstv.at[j0 + b]], ssems[b], add=True)
            for b in range(NBUF):
                pltpu.make_async_copy(
                    bufs[b], acc_sh.at[dstv.at[j0 + b]], ssems[b]).wait()
                pltpu.async_copy(
                    g2_hbm.at[srcv.at[j0 + NBUF + b]], bufs[b], gsems[b])
            return carry

        lax.fori_loop(0, hr // NBUF - 1, body, 0)
        j0 = hr - NBUF
        for b in range(NBUF):
            pltpu.make_async_copy(
                g2_hbm.at[srcv.at[j0 + b]], bufs[b], gsems[b]).wait()
            pltpu.async_copy(
                bufs[b], acc_sh.at[dstv.at[j0 + b]], ssems[b], add=True)
        for b in range(NBUF):
            pltpu.make_async_copy(
                bufs[b], acc_sh.at[dstv.at[j0 + b]], ssems[b]).wait()
    plsc.subcore_barrier()
    pltpu.sync_copy(
        acc_sh.at[pl.ds(s * ZROWS, ZROWS)],
        acc_hbm.at[c, pl.ds(s * ZROWS, ZROWS)],
    )


def _prop_call(*args):
    mesh = plsc.VectorSubcoreMesh(
        core_axis_name="c", subcore_axis_name="s",
        num_cores=NC, num_subcores=NS,
    )
    fn = functools.partial(
        pl.kernel,
        out_type=jax.ShapeDtypeStruct((NC, ACC_SP, HALF), jnp.float32),
        mesh=mesh,
        scratch_types=(
            [
                pltpu.VMEM((PROWS_SUB // 4, PCH), jnp.int32),
                pltpu.VMEM((PROWS_SUB // 4, PCH), jnp.int32),
            ]
            + [pltpu.VMEM((PCH, HALF), jnp.float32) for _ in range(NBUF)]
            + [pltpu.VMEM_SHARED((ACC_SP, HALF), jnp.float32)]
            + [pltpu.SemaphoreType.DMA for _ in range(2 * NBUF)]
        ),
    )(_prop_body)
    return fn(*args)


def _dense_body(x_ref, w_ref, b_ref, dinv_ref, g3_ref):
    h = lax.dot_general(
        x_ref[...], w_ref[...], (((1,), (1,)), ((), ())),
        preferred_element_type=jnp.float32,
    )
    h = h + b_ref[...]
    norm = jnp.sqrt(jnp.sum(h * h, axis=1, keepdims=True))
    g = h / jnp.maximum(norm, 1e-12) * SCALE
    g = g * dinv_ref[...]
    g3_ref[...] = g.reshape(ROW_BLK, 2, HALF)


def _dense_call(x, W1, b1r, dinv2):
    grid = N_NODES // ROW_BLK
    return pl.pallas_call(
        _dense_body,
        grid=(grid,),
        in_specs=[
            pl.BlockSpec((ROW_BLK, D_FEAT), lambda i: (i, 0)),
            pl.BlockSpec((D_FEAT, D_FEAT), lambda i: (0, 0)),
            pl.BlockSpec((1, D_FEAT), lambda i: (0, 0)),
            pl.BlockSpec((ROW_BLK, 1), lambda i: (i, 0)),
        ],
        out_specs=pl.BlockSpec((ROW_BLK, 2, HALF), lambda i: (i, 0, 0)),
        out_shape=jax.ShapeDtypeStruct((N_NODES, 2, HALF), jnp.float32),
    )(x, W1, b1r, dinv2)


def _combine_body(acc_ref, g3_ref, dinv_ref, out_ref):
    dinv = dinv_ref[...]
    out_ref[:, :HALF] = (acc_ref[0] + g3_ref[:, 0, :]) * dinv
    out_ref[:, HALF:] = (acc_ref[1] + g3_ref[:, 1, :]) * dinv


def _combine_call(acc, g3, dinv2):
    grid = N_NODES // ROW_BLK
    return pl.pallas_call(
        _combine_body,
        grid=(grid,),
        in_specs=[
            pl.BlockSpec((NC, ROW_BLK, HALF), lambda i: (0, i, 0)),
            pl.BlockSpec((ROW_BLK, 2, HALF), lambda i: (i, 0, 0)),
            pl.BlockSpec((ROW_BLK, 1), lambda i: (i, 0)),
        ],
        out_specs=pl.BlockSpec((ROW_BLK, D_FEAT), lambda i: (i, 0)),
        out_shape=jax.ShapeDtypeStruct((N_NODES, D_FEAT), jnp.float32),
    )(acc, g3, dinv2)


def kernel(x, edge_index, W1, b1):
    n_edges = edge_index.shape[1]
    pad = E_PAD - n_edges
    src = edge_index[0]
    dst = edge_index[1]
    src_p = jnp.concatenate([src, jnp.zeros((pad,), jnp.int32)])
    dst_p = jnp.concatenate([dst, jnp.full((pad,), N_NODES, jnp.int32)])
    dst2d = dst_p.reshape(ROWS_ALL, CHUNK)
    dst2p = dst_p.reshape(PROWS_ALL, PCH)
    src2 = jnp.stack([src_p * 2, src_p * 2 + 1]).reshape(NC, PROWS_ALL, PCH)

    ones_a = jnp.ones((CHUNK, DEG_W), jnp.float32)
    zeros_a = jnp.zeros((ZROWS, DEG_W), jnp.float32)
    zeros_c = jnp.zeros((ZROWS, HALF), jnp.float32)

    cnt = _deg_call(dst2d, ones_a, zeros_a)
    deg = cnt[0, :N_NODES, 0] + cnt[1, :N_NODES, 0] + 1.0
    dinv2 = lax.rsqrt(deg)[:, None]

    g3 = _dense_call(x, W1, b1[None, :], dinv2)
    g2 = g3.reshape(2 * N_NODES, HALF)

    acc = _prop_call(g2, src2, dst2p, zeros_c)
    return _combine_call(acc[:, :N_NODES], g3, dinv2)
